# baseline (speedup 1.0000x reference)
import jax
import jax.numpy as jnp
from jax.experimental import pallas as pl


def _voxel_body(x_ref, o_ref):
    x = x_ref[...]
    mn = jnp.min(x, axis=1, keepdims=True)
    mx = jnp.max(x, axis=1, keepdims=True)
    bin_width = (mx - mn) / 40.0
    o_ref[...] = jnp.floor((x - mn) / bin_width)


def kernel(point_cloud):
    b, n, c = point_cloud.shape
    return pl.pallas_call(
        _voxel_body,
        grid=(b,),
        in_specs=[pl.BlockSpec((1, n, c), lambda i: (i, 0, 0))],
        out_specs=pl.BlockSpec((1, n, c), lambda i: (i, 0, 0)),
        out_shape=jax.ShapeDtypeStruct((b, n, c), jnp.float32),
    )(point_cloud)
